# baseline (speedup 1.0000x reference)
import functools

import jax
import jax.numpy as jnp
from jax import lax
from jax.experimental import pallas as pl
from jax.experimental.pallas import tpu as pltpu
from jax.experimental.pallas import tpu_sc as plsc

_S = 64.0
_M3 = 0.35

_B = 1024
_C = 100000
_BR = 16
_NB = _B // _BR

_NC = 2
_NS = 16
_L = 16
_NW = _NC * _NS
_PER_W = _B // _NW


def _scale_body(x_ref, o_ref):
    o_ref[...] = x_ref[...] * _S


def _scale(logits):
    return pl.pallas_call(
        _scale_body,
        grid=(_NB,),
        in_specs=[pl.BlockSpec((_BR, _C), lambda i: (i, 0))],
        out_specs=pl.BlockSpec((_BR, _C), lambda i: (i, 0)),
        out_shape=jax.ShapeDtypeStruct((_B, _C), jnp.float32),
    )(logits)


_sc_mesh = plsc.VectorSubcoreMesh(
    core_axis_name="c", subcore_axis_name="s", num_cores=_NC, num_subcores=_NS
)


@functools.partial(
    pl.kernel,
    mesh=_sc_mesh,
    scratch_types=[
        pltpu.VMEM((_PER_W,), jnp.int32),
        pltpu.VMEM((_PER_W,), jnp.float32),
        pltpu.SemaphoreType.DMA,
    ],
)
def _sc_fix(logits_flat, out_flat, labels_hbm, idx_v, vals_v, sem):
    wid = lax.axis_index("s") * _NC + lax.axis_index("c")
    base = wid * _PER_W
    pltpu.sync_copy(labels_hbm.at[pl.ds(base, _PER_W)], idx_v)
    for j in range(_PER_W // _L):
        lab = idx_v[pl.ds(j * _L, _L)]
        row = base + j * _L + lax.iota(jnp.int32, _L)
        idx_v[pl.ds(j * _L, _L)] = row * _C + lab
    pltpu.async_copy(logits_flat.at[idx_v], vals_v, sem).wait()
    for j in range(_PER_W // _L):
        v = vals_v[pl.ds(j * _L, _L)]
        vals_v[pl.ds(j * _L, _L)] = (v - _M3) * _S
    pltpu.async_copy(vals_v, out_flat.at[idx_v], sem).wait()


@jax.jit
def _combined(logits, labels):
    scaled = _scale(logits)
    buf = jax.new_ref(scaled.reshape(_B * _C))
    _sc_fix(logits.reshape(_B * _C), buf, labels)
    return buf[...].reshape(_B, _C)


def kernel(logits, labels):
    return _combined(logits, labels.astype(jnp.int32))


# speedup vs baseline: 1.0015x; 1.0015x over previous
import functools

import jax
import jax.numpy as jnp
from jax import lax
from jax.experimental import pallas as pl
from jax.experimental.pallas import tpu as pltpu
from jax.experimental.pallas import tpu_sc as plsc

_S = 64.0
_M3 = 0.35

_B = 1024
_C = 100000
_BR = 16
_NB = _B // _BR

_NC = 2
_NS = 16
_L = 16
_NW = _NC * _NS
_PER_W = _B // _NW


def _scale_body(x_ref, o_ref):
    o_ref[...] = x_ref[...] * _S


def _scale(logits):
    return pl.pallas_call(
        _scale_body,
        grid=(_NB,),
        in_specs=[pl.BlockSpec((_BR, _C), lambda i: (i, 0))],
        out_specs=pl.BlockSpec((_BR, _C), lambda i: (i, 0)),
        out_shape=jax.ShapeDtypeStruct((_B, _C), jnp.float32),
    )(logits)


_sc_mesh = plsc.VectorSubcoreMesh(
    core_axis_name="c", subcore_axis_name="s", num_cores=_NC, num_subcores=_NS
)


@functools.partial(
    pl.kernel,
    mesh=_sc_mesh,
    scratch_types=[
        pltpu.VMEM((_PER_W,), jnp.int32),
        pltpu.VMEM((_PER_W,), jnp.float32),
        pltpu.SemaphoreType.DMA,
    ],
)
def _sc_fix(logits_flat, out_flat, labels_hbm, idx_v, vals_v, sem):
    wid = lax.axis_index("s") * _NC + lax.axis_index("c")
    base = wid * _PER_W
    pltpu.sync_copy(labels_hbm.at[pl.ds(base, _PER_W)], idx_v)
    for j in range(_PER_W // _L):
        lab = idx_v[pl.ds(j * _L, _L)]
        row = base + j * _L + lax.iota(jnp.int32, _L)
        idx_v[pl.ds(j * _L, _L)] = row * _C + lab
    pltpu.async_copy(logits_flat.at[idx_v], vals_v, sem).wait()
    for j in range(_PER_W // _L):
        v = vals_v[pl.ds(j * _L, _L)]
        vals_v[pl.ds(j * _L, _L)] = (v - _M3) * _S
    pltpu.async_copy(vals_v, out_flat.at[idx_v], sem).wait()


@jax.jit
def _combined(logits, labels):
    scaled = _scale(logits)
    buf = jax.new_ref(scaled.reshape(_B * _C))
    _sc_fix(logits.reshape(_B * _C), buf, labels)
    return jax.freeze(buf).reshape(_B, _C)


def kernel(logits, labels):
    return _combined(logits, labels.astype(jnp.int32))


# speedup vs baseline: 2.6529x; 2.6490x over previous
import functools

import jax
import jax.numpy as jnp
from jax import lax
from jax.experimental import pallas as pl
from jax.experimental.pallas import tpu as pltpu
from jax.experimental.pallas import tpu_sc as plsc

_S = 64.0
_M3 = 0.35

_B = 1024
_C = 100000
_BR = 32
_NB = _B // _BR

_NC = 2
_NS = 16
_L = 16
_NW = _NC * _NS
_PER_W = _B // _NW
_W = 32


def _scale_body(x_ref, o_ref):
    o_ref[...] = x_ref[...] * _S


def _scale(logits):
    return pl.pallas_call(
        _scale_body,
        grid=(_NB,),
        in_specs=[pl.BlockSpec((_BR, _C), lambda i: (i, 0))],
        out_specs=pl.BlockSpec((_BR, _C), lambda i: (i, 0)),
        out_shape=jax.ShapeDtypeStruct((_B, _C), jnp.float32),
    )(logits)


_sc_mesh = plsc.VectorSubcoreMesh(
    core_axis_name="c", subcore_axis_name="s", num_cores=_NC, num_subcores=_NS
)


@functools.partial(
    pl.kernel,
    mesh=_sc_mesh,
    scratch_types=[
        pltpu.VMEM((_PER_W,), jnp.int32),
        pltpu.VMEM((_W,), jnp.float32),
        pltpu.VMEM((_W,), jnp.float32),
    ],
)
def _sc_fix(logits_hbm, out_buf, labels_hbm, lab_v, win_in, win_out):
    wid = lax.axis_index("s") * _NC + lax.axis_index("c")
    base = wid * _PER_W
    pltpu.sync_copy(labels_hbm.at[pl.ds(base, _PER_W)], lab_v)
    lanes = lax.iota(jnp.int32, _L)
    for r in range(_PER_W):
        chunk = lab_v[pl.ds((r // _L) * _L, _L)]
        col = chunk[r % _L]
        cs = (col // _W) * _W
        row = base + r
        pltpu.sync_copy(logits_hbm.at[row, pl.ds(cs, _W)], win_in)
        tgt = col - cs
        for j in range(_W // _L):
            x = win_in[pl.ds(j * _L, _L)]
            hit = (lanes + (j * _L)) == tgt
            win_out[pl.ds(j * _L, _L)] = jnp.where(hit, (x - _M3) * _S, x * _S)
        pltpu.sync_copy(win_out, out_buf.at[row, pl.ds(cs, _W)])


@jax.jit
def _combined(logits, labels):
    scaled = _scale(logits)
    buf = jax.new_ref(scaled)
    _sc_fix(logits, buf, labels)
    return jax.freeze(buf)


def kernel(logits, labels):
    return _combined(logits, labels.astype(jnp.int32))


# speedup vs baseline: 2.6947x; 1.0158x over previous
import functools

import jax
import jax.numpy as jnp
from jax import lax
from jax.experimental import pallas as pl
from jax.experimental.pallas import tpu as pltpu
from jax.experimental.pallas import tpu_sc as plsc

_S = 64.0
_M3 = 0.35

_B = 1024
_C = 100000
_BR = 32
_NB = _B // _BR

_NC = 2
_NS = 16
_L = 16
_NW = _NC * _NS
_PER_W = _B // _NW
_W = 32


def _scale_body(x_ref, o_ref):
    o_ref[...] = x_ref[...] * _S


def _scale(logits):
    return pl.pallas_call(
        _scale_body,
        grid=(_NB,),
        in_specs=[pl.BlockSpec((_BR, _C), lambda i: (i, 0))],
        out_specs=pl.BlockSpec((_BR, _C), lambda i: (i, 0)),
        out_shape=jax.ShapeDtypeStruct((_B, _C), jnp.float32),
    )(logits)


_sc_mesh = plsc.VectorSubcoreMesh(
    core_axis_name="c", subcore_axis_name="s", num_cores=_NC, num_subcores=_NS
)


@functools.partial(
    pl.kernel,
    mesh=_sc_mesh,
    scratch_types=[
        pltpu.VMEM((_PER_W,), jnp.int32),
        pltpu.VMEM((_PER_W, _W), jnp.float32),
        pltpu.SemaphoreType.DMA,
    ],
)
def _sc_fix(logits_hbm, out_buf, labels_hbm, lab_v, wins, sem):
    wid = lax.axis_index("s") * _NC + lax.axis_index("c")
    base = wid * _PER_W
    pltpu.sync_copy(labels_hbm.at[pl.ds(base, _PER_W)], lab_v)
    lanes = lax.iota(jnp.int32, _L)
    chunks = [lab_v[pl.ds(j * _L, _L)] for j in range(_PER_W // _L)]
    cols = [chunks[r // _L][r % _L] for r in range(_PER_W)]
    starts = [(col // _W) * _W for col in cols]
    gathers = [
        pltpu.async_copy(
            logits_hbm.at[base + r, pl.ds(starts[r], _W)], wins.at[r], sem
        )
        for r in range(_PER_W)
    ]
    for g in gathers:
        g.wait()
    for r in range(_PER_W):
        tgt = cols[r] - starts[r]
        for j in range(_W // _L):
            x = wins[r, pl.ds(j * _L, _L)]
            hit = (lanes + (j * _L)) == tgt
            wins[r, pl.ds(j * _L, _L)] = jnp.where(hit, (x - _M3) * _S, x * _S)
    scatters = [
        pltpu.async_copy(
            wins.at[r], out_buf.at[base + r, pl.ds(starts[r], _W)], sem
        )
        for r in range(_PER_W)
    ]
    for s in scatters:
        s.wait()


@jax.jit
def _combined(logits, labels):
    scaled = _scale(logits)
    buf = jax.new_ref(scaled)
    _sc_fix(logits, buf, labels)
    return jax.freeze(buf)


def kernel(logits, labels):
    return _combined(logits, labels.astype(jnp.int32))


# speedup vs baseline: 9.6788x; 3.5918x over previous
import functools

import jax
import jax.numpy as jnp
from jax import lax
from jax.experimental import pallas as pl
from jax.experimental.pallas import tpu as pltpu
from jax.experimental.pallas import tpu_sc as plsc

_S = 64.0
_M3 = 0.35

_B = 1024
_C = 100000
_BR = 2000
_NB = _C // _BR

_NC = 2
_NS = 16
_L = 16
_NW = _NC * _NS
_PER_W = _B // _NW


def _scale_body(x_ref, o_ref):
    o_ref[...] = x_ref[...] * _S


def _scale_t(logits_t):
    return pl.pallas_call(
        _scale_body,
        grid=(_NB,),
        in_specs=[pl.BlockSpec((_BR, _B), lambda i: (i, 0))],
        out_specs=pl.BlockSpec((_BR, _B), lambda i: (i, 0)),
        out_shape=jax.ShapeDtypeStruct((_C, _B), jnp.float32),
    )(logits_t)


_sc_mesh = plsc.VectorSubcoreMesh(
    core_axis_name="c", subcore_axis_name="s", num_cores=_NC, num_subcores=_NS
)


@functools.partial(
    pl.kernel,
    mesh=_sc_mesh,
    scratch_types=[
        pltpu.VMEM((_PER_W,), jnp.int32),
        pltpu.VMEM((_PER_W, _PER_W), jnp.float32),
        pltpu.SemaphoreType.DMA,
    ],
)
def _sc_fix_t(logits_t, out_buf, labels_hbm, lab_v, wins, sem):
    wid = lax.axis_index("s") * _NC + lax.axis_index("c")
    base = wid * _PER_W
    pltpu.sync_copy(labels_hbm.at[pl.ds(base, _PER_W)], lab_v)
    chunks = [lab_v[pl.ds(j * _L, _L)] for j in range(_PER_W // _L)]
    rows = [chunks[r // _L][r % _L] for r in range(_PER_W)]
    gathers = [
        pltpu.async_copy(
            logits_t.at[rows[r], pl.ds(base, _PER_W)], wins.at[r], sem
        )
        for r in range(_PER_W)
    ]
    for g in gathers:
        g.wait()
    for r in range(_PER_W):
        for j in range(_PER_W // _L):
            x = wins[r, pl.ds(j * _L, _L)]
            hit = chunks[j] == rows[r]
            wins[r, pl.ds(j * _L, _L)] = jnp.where(hit, (x - _M3) * _S, x * _S)
    scatters = [
        pltpu.async_copy(
            wins.at[r], out_buf.at[rows[r], pl.ds(base, _PER_W)], sem
        )
        for r in range(_PER_W)
    ]
    for s in scatters:
        s.wait()


@jax.jit
def _combined(logits, labels):
    logits_t = logits.T
    scaled_t = _scale_t(logits_t)
    buf = jax.new_ref(scaled_t)
    _sc_fix_t(logits_t, buf, labels)
    return jax.freeze(buf).T


def kernel(logits, labels):
    return _combined(logits, labels.astype(jnp.int32))
